# Initial kernel scaffold; baseline (speedup 1.0000x reference)
import jax
import jax.numpy as jnp
from jax.experimental import pallas as pl


def kernel(query, key):
    raise NotImplementedError("write your pallas kernel here")


# speedup vs baseline: 19.1409x; 19.1409x over previous
import functools

import jax
import jax.numpy as jnp
from jax.experimental import pallas as pl

QK_DIM = 32
TOPK = 4
SCALE = QK_DIM ** (-0.5)


def _routing_kernel(q_ref, k_ref, w_ref, i_ref, *, n_keys):
    q = q_ref[0]
    k = k_ref[0]
    logits = jax.lax.dot_general(
        q * SCALE, k,
        dimension_numbers=(((1,), (1,)), ((), ())),
        preferred_element_type=jnp.float32,
    )

    rb = logits.shape[0]
    col = jax.lax.broadcasted_iota(jnp.int32, (rb, n_keys), 1)

    x = logits
    vals = []
    idxs = []
    for _ in range(TOPK):
        m = jnp.max(x, axis=1, keepdims=True)
        cand = jnp.where(x == m, col, n_keys)
        a = jnp.min(cand, axis=1, keepdims=True)
        vals.append(m)
        idxs.append(a)
        x = jnp.where(col == a, -jnp.inf, x)

    topv = jnp.concatenate(vals, axis=1)
    topi = jnp.concatenate(idxs, axis=1)

    e = jnp.exp(topv - vals[0])
    w = e / jnp.sum(e, axis=1, keepdims=True)

    w_ref[0] = w
    i_ref[0] = topi


def kernel(query, key):
    b, n, d = query.shape
    rb = 256
    grid = (b, n // rb)
    f = functools.partial(_routing_kernel, n_keys=n)
    w, i = pl.pallas_call(
        f,
        grid=grid,
        in_specs=[
            pl.BlockSpec((1, rb, d), lambda bi, ri: (bi, ri, 0)),
            pl.BlockSpec((1, n, d), lambda bi, ri: (bi, 0, 0)),
        ],
        out_specs=[
            pl.BlockSpec((1, rb, TOPK), lambda bi, ri: (bi, ri, 0)),
            pl.BlockSpec((1, rb, TOPK), lambda bi, ri: (bi, ri, 0)),
        ],
        out_shape=[
            jax.ShapeDtypeStruct((b, n, TOPK), jnp.float32),
            jax.ShapeDtypeStruct((b, n, TOPK), jnp.int32),
        ],
    )(query, key)
    return (w, i)


# speedup vs baseline: 25.1849x; 1.3158x over previous
import functools

import jax
import jax.numpy as jnp
from jax.experimental import pallas as pl

QK_DIM = 32
TOPK = 4
SCALE = QK_DIM ** (-0.5)


def _routing_kernel(q_ref, k_ref, w_ref, i_ref, *, n_keys):
    q = q_ref[0]
    k = k_ref[0]
    logits = jax.lax.dot_general(
        q * SCALE, k,
        dimension_numbers=(((1,), (1,)), ((), ())),
        preferred_element_type=jnp.float32,
    )

    rb = logits.shape[0]
    colf = jax.lax.broadcasted_iota(jnp.int32, (rb, n_keys), 1).astype(
        jnp.float32)

    x = logits
    vals = []
    idxs = []
    for _ in range(TOPK):
        m = jnp.max(x, axis=1, keepdims=True)
        cand = jnp.where(x == m, colf, float(n_keys))
        a = jnp.min(cand, axis=1, keepdims=True)
        vals.append(m)
        idxs.append(a)
        x = jnp.where(colf == a, -jnp.inf, x)

    topv = jnp.concatenate(vals, axis=1)
    topi = jnp.concatenate(idxs, axis=1).astype(jnp.int32)

    e = jnp.exp(topv - vals[0])
    w = e / jnp.sum(e, axis=1, keepdims=True)

    w_ref[0] = w
    i_ref[0] = topi


def kernel(query, key):
    b, n, d = query.shape
    rb = 256
    grid = (b, n // rb)
    f = functools.partial(_routing_kernel, n_keys=n)
    w, i = pl.pallas_call(
        f,
        grid=grid,
        in_specs=[
            pl.BlockSpec((1, rb, d), lambda bi, ri: (bi, ri, 0)),
            pl.BlockSpec((1, n, d), lambda bi, ri: (bi, 0, 0)),
        ],
        out_specs=[
            pl.BlockSpec((1, rb, TOPK), lambda bi, ri: (bi, ri, 0)),
            pl.BlockSpec((1, rb, TOPK), lambda bi, ri: (bi, ri, 0)),
        ],
        out_shape=[
            jax.ShapeDtypeStruct((b, n, TOPK), jnp.float32),
            jax.ShapeDtypeStruct((b, n, TOPK), jnp.int32),
        ],
    )(query, key)
    return (w, i)


# speedup vs baseline: 32.9244x; 1.3073x over previous
import functools

import jax
import jax.numpy as jnp
from jax.experimental import pallas as pl

QK_DIM = 32
TOPK = 4
SCALE = QK_DIM ** (-0.5)


def _routing_kernel(q_ref, k_ref, w_ref, i_ref, *, n_keys):
    q = q_ref[0]
    k = k_ref[0]
    x = jax.lax.dot_general(
        q * SCALE, k,
        dimension_numbers=(((1,), (1,)), ((), ())),
        preferred_element_type=jnp.float32,
    )

    rb = x.shape[0]
    colf = jax.lax.broadcasted_iota(jnp.int32, (rb, n_keys), 1).astype(
        jnp.float32)
    big = float(n_keys)

    vals = []
    idxs = []
    for j in range(TOPK):
        m = jnp.max(x, axis=1, keepdims=True)
        eq = x == m
        cand = jnp.where(eq, colf, big)
        a = jnp.min(cand, axis=1, keepdims=True)
        vals.append(m)
        idxs.append(a)
        if j < TOPK - 1:
            x = jnp.where(eq, -jnp.inf, x)

    topv = jnp.concatenate(vals, axis=1)
    topi = jnp.concatenate(idxs, axis=1).astype(jnp.int32)

    e = jnp.exp(topv - vals[0])
    w = e / jnp.sum(e, axis=1, keepdims=True)

    w_ref[0] = w
    i_ref[0] = topi


def kernel(query, key):
    b, n, d = query.shape
    rb = 1024
    grid = (b, n // rb)
    f = functools.partial(_routing_kernel, n_keys=n)
    w, i = pl.pallas_call(
        f,
        grid=grid,
        in_specs=[
            pl.BlockSpec((1, rb, d), lambda bi, ri: (bi, ri, 0)),
            pl.BlockSpec((1, n, d), lambda bi, ri: (bi, 0, 0)),
        ],
        out_specs=[
            pl.BlockSpec((1, rb, TOPK), lambda bi, ri: (bi, ri, 0)),
            pl.BlockSpec((1, rb, TOPK), lambda bi, ri: (bi, ri, 0)),
        ],
        out_shape=[
            jax.ShapeDtypeStruct((b, n, TOPK), jnp.float32),
            jax.ShapeDtypeStruct((b, n, TOPK), jnp.int32),
        ],
    )(query, key)
    return (w, i)
